# baseline (speedup 1.0000x reference)
import functools

import jax
import jax.numpy as jnp
from jax import lax
from jax.experimental import pallas as pl
from jax.experimental.pallas import tpu as pltpu
from jax.experimental.pallas import tpu_sc as plsc


def _make_sc_broadcast(B, L, D, NC, NS):
    NW = NC * NS
    rows_per_w = B // NW
    row_words = L * D
    padded_row = L * max(D, 128) * 4
    rep = 1
    for cand in range(min(rows_per_w, (448 * 1024) // padded_row), 0, -1):
        if rows_per_w % cand == 0:
            rep = cand
            break
    n_dma = rows_per_w // rep

    mesh = plsc.VectorSubcoreMesh(core_axis_name="c", subcore_axis_name="s")

    @functools.partial(
        pl.kernel,
        mesh=mesh,
        out_type=jax.ShapeDtypeStruct((B, L, D), jnp.float32),
        scratch_types=[
            pltpu.VMEM((rep, L, D), jnp.float32),
            pltpu.SemaphoreType.DMA,
        ],
    )
    def k(table_hbm, out_hbm, buf, sem):
        wid = lax.axis_index("s") * NC + lax.axis_index("c")
        stage = [pltpu.async_copy(table_hbm, buf.at[r], sem) for r in range(rep)]
        for c in stage:
            c.wait()
        copies = [
            pltpu.async_copy(
                buf, out_hbm.at[pl.ds((i * NW + wid) * rep, rep)], sem
            )
            for i in range(n_dma)
        ]
        for c in copies:
            c.wait()

    return k


def kernel(x, pos_table):
    B, L = x.shape
    D = pos_table.shape[1]
    info = plsc.get_sparse_core_info()
    NC, NS = info.num_cores, info.num_subcores
    k = _make_sc_broadcast(B, L, D, NC, NS)
    return k(pos_table[:L])
